# baseline (speedup 1.0000x reference)
import functools

import jax
import jax.numpy as jnp
from jax import lax
from jax.experimental import pallas as pl
from jax.experimental.pallas import tpu as pltpu
from jax.experimental.pallas import tpu_sc as plsc

N = 50000
H = 128
NP = 51200
RPT = NP // 16
EPAD = 819200
EB = EPAD // 128
BLK = 1600
NBLK = NP // BLK
NREAL = float(N)

_MESH = plsc.VectorSubcoreMesh(core_axis_name="c", subcore_axis_name="s")
_SC_PARAMS = pltpu.CompilerParams(use_tc_tiling_on_sc=False)
_Z16 = functools.partial(jnp.zeros, (16,), jnp.float32)


def _zero_fill(rows, nrow):
    def body(i, carry):
        rows[i, pl.ds(0, 16)] = _Z16()
        rows[i, pl.ds(16, 16)] = _Z16()
        return carry
    lax.fori_loop(0, nrow, body, 0)


def _zero_acc(rows, acc, row0):
    for k in range(6):
        pltpu.sync_copy(rows.at[pl.ds(0, 512)],
                        acc.at[pl.ds(row0 + k * 512, 512)])
    pltpu.sync_copy(rows.at[pl.ds(0, 128)], acc.at[pl.ds(row0 + 3072, 128)])


def _edge_blocks(npair, srcb_view, dstb, sidx, didx, rows, acc, isem, gsem, ssem,
                 rbase_fn, gather_ref):
    def body(i, carry):
        rb0 = rbase_fn(2 * i)
        rb1 = rbase_fn(2 * i + 1)
        pltpu.sync_copy(srcb_view.at[pl.ds(rb0, 2)], sidx.at[0])
        pltpu.sync_copy(dstb.at[pl.ds(rb0, 2)], didx.at[0])
        g0 = [pltpu.async_copy(gather_ref.at[sidx.at[0, j]],
                               rows.at[pl.ds(j * 128, 128)], gsem)
              for j in range(2)]
        i0 = pltpu.async_copy(srcb_view.at[pl.ds(rb1, 2)], sidx.at[1], isem)
        i1 = pltpu.async_copy(dstb.at[pl.ds(rb1, 2)], didx.at[1], isem)
        for d_ in g0:
            d_.wait()
        s0 = [pltpu.async_copy(rows.at[pl.ds(j * 128, 128)],
                               acc.at[didx.at[0, j]], ssem, add=True)
              for j in range(2)]
        i0.wait()
        i1.wait()
        g1 = [pltpu.async_copy(gather_ref.at[sidx.at[1, j]],
                               rows.at[pl.ds(256 + j * 128, 128)], gsem)
              for j in range(2)]
        for d_ in g1:
            d_.wait()
        s1 = [pltpu.async_copy(rows.at[pl.ds(256 + j * 128, 128)],
                               acc.at[didx.at[1, j]], ssem, add=True)
              for j in range(2)]
        for d_ in s0:
            d_.wait()
        for d_ in s1:
            d_.wait()
        return carry
    lax.fori_loop(0, npair, body, 0)


def _prop_body(uflat, srcq, dstb, sout, acc, sidx, didx, rows, isem, gsem, ssem):
    c = lax.axis_index("c")
    s = lax.axis_index("s")
    row0 = s * RPT
    rpt_idx = (EPAD // 16) // 128

    for r in range(2):
        q = 2 * r + c
        _zero_fill(rows, 512)
        _zero_acc(rows, acc, row0)
        plsc.subcore_barrier()
        _edge_blocks(rpt_idx // 4, srcq.at[q], dstb, sidx, didx, rows, acc,
                     isem, gsem, ssem, lambda b: s * rpt_idx + b * 2, uflat)
        plsc.subcore_barrier()
        pltpu.sync_copy(acc.at[pl.ds(row0, RPT)], sout.at[q, pl.ds(row0, RPT)])
        plsc.subcore_barrier()


_prop_call = pl.kernel(
    _prop_body,
    compiler_params=_SC_PARAMS,
    out_type=jax.ShapeDtypeStruct((4, NP, 32), jnp.float32),
    mesh=_MESH,
    scratch_types=[
        pltpu.VMEM_SHARED((NP, 32), jnp.float32),
        pltpu.VMEM((2, 2, 128), jnp.int32),
        pltpu.VMEM((2, 2, 128), jnp.int32),
        pltpu.VMEM((512, 32), jnp.float32),
        pltpu.SemaphoreType.DMA,
        pltpu.SemaphoreType.DMA,
        pltpu.SemaphoreType.DMA,
    ],
)


def _xprop_body(u32, srcb, dstb, sout, acc, sidx, didx, rows, isem, gsem, ssem):
    c = lax.axis_index("c")
    s = lax.axis_index("s")
    row0 = s * RPT
    rpt_idx = (EPAD // 32) // 128
    _zero_fill(rows, 512)
    _zero_acc(rows, acc, row0)
    plsc.subcore_barrier()
    _edge_blocks(rpt_idx // 4, srcb, dstb, sidx, didx, rows, acc, isem, gsem,
                 ssem, lambda b: (c * 16 + s) * rpt_idx + b * 2, u32)
    plsc.subcore_barrier()
    pltpu.sync_copy(acc.at[pl.ds(row0, RPT)], sout.at[c, pl.ds(row0, RPT)])


_xprop_call = pl.kernel(
    _xprop_body,
    compiler_params=_SC_PARAMS,
    out_type=jax.ShapeDtypeStruct((2, NP, 32), jnp.float32),
    mesh=_MESH,
    scratch_types=[
        pltpu.VMEM_SHARED((NP, 32), jnp.float32),
        pltpu.VMEM((2, 2, 128), jnp.int32),
        pltpu.VMEM((2, 2, 128), jnp.int32),
        pltpu.VMEM((512, 32), jnp.float32),
        pltpu.SemaphoreType.DMA,
        pltpu.SemaphoreType.DMA,
        pltpu.SemaphoreType.DMA,
    ],
)


def _deg_body(srcb, degp, accd, sidx, ones, zbufd, ssem):
    c = lax.axis_index("c")
    s = lax.axis_index("s")
    row0 = s * RPT
    for j in range(8):
        ones[pl.ds(j * 16, 16)] = jnp.ones((16,), jnp.float32)

    def zb(i, carry):
        zbufd[pl.ds(i * 16, 16)] = _Z16()
        return carry
    lax.fori_loop(0, 200, zb, 0)
    pltpu.sync_copy(zbufd.at[pl.ds(0, RPT)], accd.at[pl.ds(row0, RPT)])
    plsc.subcore_barrier()
    rpt_idx = (EPAD // 32) // 128

    def body(b, carry):
        rbase = (c * 16 + s) * rpt_idx + b * 8
        pltpu.sync_copy(srcb.at[pl.ds(rbase, 8)], sidx)
        sds = [pltpu.async_copy(ones, accd.at[sidx.at[j]], ssem, add=True)
               for j in range(8)]
        for d_ in sds:
            d_.wait()
        return carry
    lax.fori_loop(0, rpt_idx // 8, body, 0)
    plsc.subcore_barrier()
    pltpu.sync_copy(accd.at[pl.ds(row0, RPT)], degp.at[pl.ds(c * NP + row0, RPT)])


_deg_call = pl.kernel(
    _deg_body,
    compiler_params=_SC_PARAMS,
    out_type=jax.ShapeDtypeStruct((2 * NP,), jnp.float32),
    mesh=_MESH,
    scratch_types=[
        pltpu.VMEM_SHARED((NP,), jnp.float32),
        pltpu.VMEM((8, 128), jnp.int32),
        pltpu.VMEM((128,), jnp.float32),
        pltpu.VMEM((3200,), jnp.float32),
        pltpu.SemaphoreType.DMA,
    ],
)


def _rowspec(w):
    return pl.BlockSpec((BLK, w), lambda j: (j, 0))


def _chunkspec(part):
    return pl.BlockSpec((1, BLK, 32), lambda j, _p=part: (_p, j, 0))


def _fullspec(shape):
    nd = len(shape)
    return pl.BlockSpec(shape, lambda j: (0,) * nd)


def _ux_body(x_ref, dv_ref, o_ref):
    o_ref[...] = x_ref[...] * dv_ref[...]


def _ux(x32, dinv):
    return pl.pallas_call(
        _ux_body, grid=(NBLK,),
        in_specs=[_rowspec(32), _rowspec(1)],
        out_specs=_rowspec(32),
        out_shape=jax.ShapeDtypeStruct((NP, 32), jnp.float32),
    )(x32, dinv)


def _u2w3_body(a_ref, b_ref, d2_ref, o_ref):
    o_ref[...] = -(a_ref[0] + b_ref[0]) * d2_ref[...]


def _u2w3(S1, dinv2):
    return pl.pallas_call(
        _u2w3_body, grid=(NBLK,),
        in_specs=[_chunkspec(0), _chunkspec(1), _rowspec(1)],
        out_specs=_rowspec(32),
        out_shape=jax.ShapeDtypeStruct((NP, 32), jnp.float32),
    )(S1, S1, dinv2)


def _u3w3_body(a_ref, b_ref, u_ref, d2_ref, o_ref):
    o_ref[...] = -2.0 * (a_ref[0] + b_ref[0]) * d2_ref[...] - u_ref[...]


def _u3w3(S2, uxv, dinv2):
    return pl.pallas_call(
        _u3w3_body, grid=(NBLK,),
        in_specs=[_chunkspec(0), _chunkspec(1), _rowspec(32), _rowspec(1)],
        out_specs=_rowspec(32),
        out_shape=jax.ShapeDtypeStruct((NP, 32), jnp.float32),
    )(S2, S2, uxv, dinv2)


def _stats_of(out, j):
    rid = lax.broadcasted_iota(jnp.int32, (BLK, 1), 0) + j * BLK
    m = (rid < N).astype(jnp.float32)
    om = out * m
    s0 = jnp.sum(om, axis=0)
    s1 = jnp.sum(om * out, axis=0)
    return jnp.concatenate(
        [s0[None], s1[None], jnp.zeros((6, H), jnp.float32)], axis=0)[None]


def _l1a_body(x_ref, s1a, s1b, s2a, s2b, s3a, s3b, dv, a0, a1, a2, a3, bv,
              o_ref, st_ref):
    j = pl.program_id(0)
    dvv = dv[...]
    t0 = x_ref[...]
    t1 = -(s1a[0] + s1b[0]) * dvv
    t2 = -2.0 * (s2a[0] + s2b[0]) * dvv - t0
    t3 = -2.0 * (s3a[0] + s3b[0]) * dvv - t1
    out = jnp.dot(t0, a0[...], preferred_element_type=jnp.float32)
    out += jnp.dot(t1, a1[...], preferred_element_type=jnp.float32)
    out += jnp.dot(t2, a2[...], preferred_element_type=jnp.float32)
    out += jnp.dot(t3, a3[...], preferred_element_type=jnp.float32)
    out += bv[...]
    o_ref[...] = out
    st_ref[...] = _stats_of(out, j)


def _l1a(x32, S1, S2, S3, dinv, a0, a1, a2, a3, bvec):
    return pl.pallas_call(
        _l1a_body, grid=(NBLK,),
        in_specs=[_rowspec(32),
                  _chunkspec(0), _chunkspec(1),
                  _chunkspec(0), _chunkspec(1),
                  _chunkspec(0), _chunkspec(1),
                  _rowspec(1),
                  _fullspec((32, H)), _fullspec((32, H)),
                  _fullspec((32, H)), _fullspec((32, H)),
                  _fullspec((1, H))],
        out_specs=[_rowspec(H), pl.BlockSpec((1, 8, H), lambda j: (j, 0, 0))],
        out_shape=[jax.ShapeDtypeStruct((NP, H), jnp.float32),
                   jax.ShapeDtypeStruct((NBLK, 8, H), jnp.float32)],
    )(x32, S1, S1, S2, S2, S3, S3, dinv, a0, a1, a2, a3, bvec)


def _lbn_body(o_ref, st_ref, g_ref, be_ref, dv_ref, h_ref, u_ref, *, slope):
    st = jnp.sum(st_ref[...], axis=0)
    mu = st[0] / NREAL
    var = st[1] / NREAL - mu * mu
    xh = (o_ref[...] - mu) / jnp.sqrt(var + 1e-5) * g_ref[0] + be_ref[0]
    h = jnp.where(xh >= 0.0, xh, xh * slope)
    h_ref[...] = h
    u_ref[...] = h * dv_ref[...]


def _lbn(out, stats, g, be, dinv, slope):
    body = functools.partial(_lbn_body, slope=slope)
    return pl.pallas_call(
        body, grid=(NBLK,),
        in_specs=[_rowspec(H),
                  pl.BlockSpec((NBLK, 8, H), lambda j: (0, 0, 0)),
                  _fullspec((1, H)), _fullspec((1, H)), _rowspec(1)],
        out_specs=[_rowspec(H), _rowspec(H)],
        out_shape=[jax.ShapeDtypeStruct((NP, H), jnp.float32),
                   jax.ShapeDtypeStruct((NP, H), jnp.float32)],
    )(out, stats, g.reshape(1, H), be.reshape(1, H), dinv)


def _cheb_out(h_ref, schunks, dv, a0, a1, a2, a3, bv):
    dvv = dv[...]
    t0 = h_ref[...]
    out = jnp.dot(t0, a0[...], preferred_element_type=jnp.float32)
    w1, w2, w3 = a1[...], a2[...], a3[...]
    t1s, t2s, t3s = [], [], []
    for q in range(4):
        t1q = -schunks[0][q][0] * dvv
        t2q = -2.0 * schunks[1][q][0] * dvv - t0[:, 32 * q:32 * q + 32]
        t3q = -2.0 * schunks[2][q][0] * dvv - t1q
        t1s.append(t1q); t2s.append(t2q); t3s.append(t3q)
    for q in range(4):
        out += jnp.dot(t1s[q], w1[32 * q:32 * q + 32, :],
                       preferred_element_type=jnp.float32)
    for q in range(4):
        out += jnp.dot(t2s[q], w2[32 * q:32 * q + 32, :],
                       preferred_element_type=jnp.float32)
    for q in range(4):
        out += jnp.dot(t3s[q], w3[32 * q:32 * q + 32, :],
                       preferred_element_type=jnp.float32)
    return out + bv[...]


def _la_body(h_ref, s10, s11, s12, s13, s20, s21, s22, s23, s30, s31, s32, s33,
             dv, a0, a1, a2, a3, bv, o_ref, st_ref):
    j = pl.program_id(0)
    out = _cheb_out(h_ref, ((s10, s11, s12, s13), (s20, s21, s22, s23),
                            (s30, s31, s32, s33)), dv, a0, a1, a2, a3, bv)
    o_ref[...] = out
    st_ref[...] = _stats_of(out, j)


def _la(h, S1, S2, S3, dinv, a0, a1, a2, a3, bvec):
    cs = [_chunkspec(q) for q in range(4)]
    return pl.pallas_call(
        _la_body, grid=(NBLK,),
        in_specs=[_rowspec(H)] + cs + cs + cs
                 + [_rowspec(1)]
                 + [_fullspec((H, H))] * 4 + [_fullspec((1, H))],
        out_specs=[_rowspec(H), pl.BlockSpec((1, 8, H), lambda j: (j, 0, 0))],
        out_shape=[jax.ShapeDtypeStruct((NP, H), jnp.float32),
                   jax.ShapeDtypeStruct((NBLK, 8, H), jnp.float32)],
    )(h, S1, S1, S1, S1, S2, S2, S2, S2, S3, S3, S3, S3,
      dinv, a0, a1, a2, a3, bvec)


def _l4_body(h_ref, s10, s11, s12, s13, s20, s21, s22, s23, s30, s31, s32, s33,
             dv, a0, a1, a2, a3, bv, wr_ref, br_ref, o_ref):
    out = _cheb_out(h_ref, ((s10, s11, s12, s13), (s20, s21, s22, s23),
                            (s30, s31, s32, s33)), dv, a0, a1, a2, a3, bv)
    nrm = jnp.sqrt(jnp.sum(out * out, axis=1, keepdims=True))
    z = out / jnp.maximum(nrm, 1e-12)
    o_ref[...] = jnp.dot(z, wr_ref[...], preferred_element_type=jnp.float32) + br_ref[...]


def _l4(h, S1, S2, S3, dinv, a0, a1, a2, a3, bvec, Wr, br):
    cs = [_chunkspec(q) for q in range(4)]
    return pl.pallas_call(
        _l4_body, grid=(NBLK,),
        in_specs=[_rowspec(H)] + cs + cs + cs
                 + [_rowspec(1)]
                 + [_fullspec((H, H))] * 4 + [_fullspec((1, H))]
                 + [_fullspec((H, 3)), _fullspec((1, 3))],
        out_specs=_rowspec(3),
        out_shape=jax.ShapeDtypeStruct((NP, 3), jnp.float32),
    )(h, S1, S1, S1, S1, S2, S2, S2, S2, S3, S3, S3, S3,
      dinv, a0, a1, a2, a3, bvec, Wr, br.reshape(1, 3))


def _u2_body(s0, s1, s2, s3, d2_ref, o_ref):
    d2 = d2_ref[...]
    for q, sr in enumerate((s0, s1, s2, s3)):
        o_ref[:, 32 * q:32 * q + 32] = -sr[0] * d2


def _u2full(S1, dinv2):
    cs = [_chunkspec(q) for q in range(4)]
    return pl.pallas_call(
        _u2_body, grid=(NBLK,),
        in_specs=cs + [_rowspec(1)],
        out_specs=_rowspec(H),
        out_shape=jax.ShapeDtypeStruct((NP, H), jnp.float32),
    )(S1, S1, S1, S1, dinv2)


def _u3_body(s0, s1, s2, s3, u_ref, d2_ref, o_ref):
    d2 = d2_ref[...]
    u = u_ref[...]
    for q, sr in enumerate((s0, s1, s2, s3)):
        o_ref[:, 32 * q:32 * q + 32] = (-2.0 * sr[0] * d2
                                        - u[:, 32 * q:32 * q + 32])


def _u3full(S2, u1, dinv2):
    cs = [_chunkspec(q) for q in range(4)]
    return pl.pallas_call(
        _u3_body, grid=(NBLK,),
        in_specs=cs + [_rowspec(H), _rowspec(1)],
        out_specs=_rowspec(H),
        out_shape=jax.ShapeDtypeStruct((NP, H), jnp.float32),
    )(S2, S2, S2, S2, u1, dinv2)


def kernel(x, edge_index, W1, b1, g1, be1, W2, b2, g2, be2, W3, b3, g3, be3,
           W4, b4, Wr, br):
    f32 = jnp.float32
    E = edge_index.shape[1]
    src = edge_index[0].astype(jnp.int32)
    dst = edge_index[1].astype(jnp.int32)
    pe = EPAD - E
    ar = jnp.arange(pe, dtype=jnp.int32)
    srcp = jnp.concatenate([src, N + (ar % 48)])
    dstp = jnp.concatenate([dst, (ar * 2557) % NP])
    srcb = srcp.reshape(EB, 128)
    dstb = dstp.reshape(EB, 128)
    srcq = ((srcp * 4)[None, :]
            + jnp.arange(4, dtype=jnp.int32)[:, None]).reshape(4, EB, 128)
    x32 = jnp.zeros((NP, 32), f32).at[:N, :3].set(x)

    degp = _deg_call(srcb).reshape(2, NP)
    deg = degp[0] + degp[1]
    valid = jnp.arange(NP) < N
    dinv = jnp.where(valid & (deg > 0),
                     1.0 / jnp.sqrt(jnp.maximum(deg, 1e-12)), 0.0)
    dinv = dinv.astype(f32)[:, None]
    dinv2 = dinv * dinv

    uxv = _ux(x32, dinv)
    S1 = _xprop_call(uxv, srcb, dstb)
    u2 = _u2w3(S1, dinv2)
    S2 = _xprop_call(u2, srcb, dstb)
    u3 = _u3w3(S2, uxv, dinv2)
    S3 = _xprop_call(u3, srcb, dstb)
    pad32 = lambda w: jnp.zeros((32, H), f32).at[:3].set(w)
    out1, st1 = _l1a(x32, S1, S2, S3, dinv,
                     pad32(W1[0]), pad32(W1[1]), pad32(W1[2]), pad32(W1[3]),
                     b1.reshape(1, H))
    h, u1 = _lbn(out1, st1, g1, be1, dinv, 0.01)

    for (W, b, g, be, slope) in ((W2, b2, g2, be2, 0.01),
                                 (W3, b3, g3, be3, 0.0)):
        S1 = _prop_call(u1.reshape(NP * 4, 32), srcq, dstb)
        u2 = _u2full(S1, dinv2)
        S2 = _prop_call(u2.reshape(NP * 4, 32), srcq, dstb)
        u3 = _u3full(S2, u1, dinv2)
        S3 = _prop_call(u3.reshape(NP * 4, 32), srcq, dstb)
        out, st = _la(h, S1, S2, S3, dinv, W[0], W[1], W[2], W[3],
                      b.reshape(1, H))
        h, u1 = _lbn(out, st, g, be, dinv, slope)

    S1 = _prop_call(u1.reshape(NP * 4, 32), srcq, dstb)
    u2 = _u2full(S1, dinv2)
    S2 = _prop_call(u2.reshape(NP * 4, 32), srcq, dstb)
    u3 = _u3full(S2, u1, dinv2)
    S3 = _prop_call(u3.reshape(NP * 4, 32), srcq, dstb)
    final = _l4(h, S1, S2, S3, dinv, W4[0], W4[1], W4[2], W4[3],
                b4.reshape(1, H), Wr, br)
    return final[:N]


# speedup vs baseline: 1.2366x; 1.2366x over previous
import functools

import jax
import jax.numpy as jnp
from jax import lax
from jax.experimental import pallas as pl
from jax.experimental.pallas import tpu as pltpu
from jax.experimental.pallas import tpu_sc as plsc

N = 50000
H = 128
NP = 51200
RPT = NP // 16
EPAD = 819200
EB = EPAD // 128
BLK = 1600
NBLK = NP // BLK
NREAL = float(N)

_MESH = plsc.VectorSubcoreMesh(core_axis_name="c", subcore_axis_name="s")
_SC_PARAMS = pltpu.CompilerParams(use_tc_tiling_on_sc=False)
_Z16 = functools.partial(jnp.zeros, (16,), jnp.float32)


def _zero_fill(rows, nrow):
    def body(i, carry):
        rows[i, pl.ds(0, 16)] = _Z16()
        rows[i, pl.ds(16, 16)] = _Z16()
        return carry
    lax.fori_loop(0, nrow, body, 0)


def _zero_acc(rows, acc, row0):
    for k in range(6):
        pltpu.sync_copy(rows.at[pl.ds(0, 512)],
                        acc.at[pl.ds(row0 + k * 512, 512)])
    pltpu.sync_copy(rows.at[pl.ds(0, 128)], acc.at[pl.ds(row0 + 3072, 128)])


def _edge_blocks(ngrp, srcb_view, dstb, sidx, didx, rows, acc, gsem, ssem,
                 rbase_fn, gather_ref):
    def body(i, carry):
        rb = rbase_fn(i)
        pltpu.sync_copy(srcb_view.at[pl.ds(rb, 8)], sidx)
        pltpu.sync_copy(dstb.at[pl.ds(rb, 8)], didx)

        def gfire(k):
            base = (k % 2) * 256
            return [pltpu.async_copy(gather_ref.at[sidx.at[2 * k + j]],
                                     rows.at[pl.ds(base + j * 128, 128)], gsem)
                    for j in range(2)]

        def sfire(k):
            base = (k % 2) * 256
            return [pltpu.async_copy(rows.at[pl.ds(base + j * 128, 128)],
                                     acc.at[didx.at[2 * k + j]], ssem, add=True)
                    for j in range(2)]

        g0 = gfire(0)
        g1 = gfire(1)
        for d_ in g0:
            d_.wait()
        s0 = sfire(0)
        for d_ in g1:
            d_.wait()
        s1 = sfire(1)
        for d_ in s0:
            d_.wait()
        g2 = gfire(2)
        for d_ in g2:
            d_.wait()
        s2 = sfire(2)
        for d_ in s1:
            d_.wait()
        g3 = gfire(3)
        for d_ in g3:
            d_.wait()
        s3 = sfire(3)
        for d_ in s2:
            d_.wait()
        for d_ in s3:
            d_.wait()
        return carry
    lax.fori_loop(0, ngrp, body, 0)


def _prop_body(uflat, srcq, dstb, sout, acc, sidx, didx, rows, gsem, ssem):
    c = lax.axis_index("c")
    s = lax.axis_index("s")
    row0 = s * RPT
    rpt_idx = (EPAD // 16) // 128

    for r in range(2):
        q = 2 * r + c
        _zero_fill(rows, 512)
        _zero_acc(rows, acc, row0)
        plsc.subcore_barrier()
        _edge_blocks(rpt_idx // 8, srcq.at[q], dstb, sidx, didx, rows, acc,
                     gsem, ssem, lambda b: s * rpt_idx + b * 8, uflat)
        plsc.subcore_barrier()
        pltpu.sync_copy(acc.at[pl.ds(row0, RPT)], sout.at[q, pl.ds(row0, RPT)])
        plsc.subcore_barrier()


_prop_call = pl.kernel(
    _prop_body,
    compiler_params=_SC_PARAMS,
    out_type=jax.ShapeDtypeStruct((4, NP, 32), jnp.float32),
    mesh=_MESH,
    scratch_types=[
        pltpu.VMEM_SHARED((NP, 32), jnp.float32),
        pltpu.VMEM((8, 128), jnp.int32),
        pltpu.VMEM((8, 128), jnp.int32),
        pltpu.VMEM((512, 32), jnp.float32),
        pltpu.SemaphoreType.DMA,
        pltpu.SemaphoreType.DMA,
    ],
)


def _xprop_body(u32, srcb, dstb, sout, acc, sidx, didx, rows, gsem, ssem):
    c = lax.axis_index("c")
    s = lax.axis_index("s")
    row0 = s * RPT
    rpt_idx = (EPAD // 32) // 128
    _zero_fill(rows, 512)
    _zero_acc(rows, acc, row0)
    plsc.subcore_barrier()
    _edge_blocks(rpt_idx // 8, srcb, dstb, sidx, didx, rows, acc, gsem,
                 ssem, lambda b: (c * 16 + s) * rpt_idx + b * 8, u32)
    plsc.subcore_barrier()
    pltpu.sync_copy(acc.at[pl.ds(row0, RPT)], sout.at[c, pl.ds(row0, RPT)])


_xprop_call = pl.kernel(
    _xprop_body,
    compiler_params=_SC_PARAMS,
    out_type=jax.ShapeDtypeStruct((2, NP, 32), jnp.float32),
    mesh=_MESH,
    scratch_types=[
        pltpu.VMEM_SHARED((NP, 32), jnp.float32),
        pltpu.VMEM((8, 128), jnp.int32),
        pltpu.VMEM((8, 128), jnp.int32),
        pltpu.VMEM((512, 32), jnp.float32),
        pltpu.SemaphoreType.DMA,
        pltpu.SemaphoreType.DMA,
    ],
)


def _deg_body(srcb, degp, accd, sidx, ones, zbufd, ssem):
    c = lax.axis_index("c")
    s = lax.axis_index("s")
    row0 = s * RPT
    for j in range(8):
        ones[pl.ds(j * 16, 16)] = jnp.ones((16,), jnp.float32)

    def zb(i, carry):
        zbufd[pl.ds(i * 16, 16)] = _Z16()
        return carry
    lax.fori_loop(0, 200, zb, 0)
    pltpu.sync_copy(zbufd.at[pl.ds(0, RPT)], accd.at[pl.ds(row0, RPT)])
    plsc.subcore_barrier()
    rpt_idx = (EPAD // 32) // 128

    def body(b, carry):
        rbase = (c * 16 + s) * rpt_idx + b * 8
        pltpu.sync_copy(srcb.at[pl.ds(rbase, 8)], sidx)
        sds = [pltpu.async_copy(ones, accd.at[sidx.at[j]], ssem, add=True)
               for j in range(8)]
        for d_ in sds:
            d_.wait()
        return carry
    lax.fori_loop(0, rpt_idx // 8, body, 0)
    plsc.subcore_barrier()
    pltpu.sync_copy(accd.at[pl.ds(row0, RPT)], degp.at[pl.ds(c * NP + row0, RPT)])


_deg_call = pl.kernel(
    _deg_body,
    compiler_params=_SC_PARAMS,
    out_type=jax.ShapeDtypeStruct((2 * NP,), jnp.float32),
    mesh=_MESH,
    scratch_types=[
        pltpu.VMEM_SHARED((NP,), jnp.float32),
        pltpu.VMEM((8, 128), jnp.int32),
        pltpu.VMEM((128,), jnp.float32),
        pltpu.VMEM((3200,), jnp.float32),
        pltpu.SemaphoreType.DMA,
    ],
)


def _rowspec(w):
    return pl.BlockSpec((BLK, w), lambda j: (j, 0))


def _chunkspec(part):
    return pl.BlockSpec((1, BLK, 32), lambda j, _p=part: (_p, j, 0))


def _fullspec(shape):
    nd = len(shape)
    return pl.BlockSpec(shape, lambda j: (0,) * nd)


def _ux_body(x_ref, dv_ref, o_ref):
    o_ref[...] = x_ref[...] * dv_ref[...]


def _ux(x32, dinv):
    return pl.pallas_call(
        _ux_body, grid=(NBLK,),
        in_specs=[_rowspec(32), _rowspec(1)],
        out_specs=_rowspec(32),
        out_shape=jax.ShapeDtypeStruct((NP, 32), jnp.float32),
    )(x32, dinv)


def _u2w3_body(a_ref, b_ref, d2_ref, o_ref):
    o_ref[...] = -(a_ref[0] + b_ref[0]) * d2_ref[...]


def _u2w3(S1, dinv2):
    return pl.pallas_call(
        _u2w3_body, grid=(NBLK,),
        in_specs=[_chunkspec(0), _chunkspec(1), _rowspec(1)],
        out_specs=_rowspec(32),
        out_shape=jax.ShapeDtypeStruct((NP, 32), jnp.float32),
    )(S1, S1, dinv2)


def _u3w3_body(a_ref, b_ref, u_ref, d2_ref, o_ref):
    o_ref[...] = -2.0 * (a_ref[0] + b_ref[0]) * d2_ref[...] - u_ref[...]


def _u3w3(S2, uxv, dinv2):
    return pl.pallas_call(
        _u3w3_body, grid=(NBLK,),
        in_specs=[_chunkspec(0), _chunkspec(1), _rowspec(32), _rowspec(1)],
        out_specs=_rowspec(32),
        out_shape=jax.ShapeDtypeStruct((NP, 32), jnp.float32),
    )(S2, S2, uxv, dinv2)


def _stats_of(out, j):
    rid = lax.broadcasted_iota(jnp.int32, (BLK, 1), 0) + j * BLK
    m = (rid < N).astype(jnp.float32)
    om = out * m
    s0 = jnp.sum(om, axis=0)
    s1 = jnp.sum(om * out, axis=0)
    return jnp.concatenate(
        [s0[None], s1[None], jnp.zeros((6, H), jnp.float32)], axis=0)[None]


def _l1a_body(x_ref, s1a, s1b, s2a, s2b, s3a, s3b, dv, a0, a1, a2, a3, bv,
              o_ref, st_ref):
    j = pl.program_id(0)
    dvv = dv[...]
    t0 = x_ref[...]
    t1 = -(s1a[0] + s1b[0]) * dvv
    t2 = -2.0 * (s2a[0] + s2b[0]) * dvv - t0
    t3 = -2.0 * (s3a[0] + s3b[0]) * dvv - t1
    out = jnp.dot(t0, a0[...], preferred_element_type=jnp.float32)
    out += jnp.dot(t1, a1[...], preferred_element_type=jnp.float32)
    out += jnp.dot(t2, a2[...], preferred_element_type=jnp.float32)
    out += jnp.dot(t3, a3[...], preferred_element_type=jnp.float32)
    out += bv[...]
    o_ref[...] = out
    st_ref[...] = _stats_of(out, j)


def _l1a(x32, S1, S2, S3, dinv, a0, a1, a2, a3, bvec):
    return pl.pallas_call(
        _l1a_body, grid=(NBLK,),
        in_specs=[_rowspec(32),
                  _chunkspec(0), _chunkspec(1),
                  _chunkspec(0), _chunkspec(1),
                  _chunkspec(0), _chunkspec(1),
                  _rowspec(1),
                  _fullspec((32, H)), _fullspec((32, H)),
                  _fullspec((32, H)), _fullspec((32, H)),
                  _fullspec((1, H))],
        out_specs=[_rowspec(H), pl.BlockSpec((1, 8, H), lambda j: (j, 0, 0))],
        out_shape=[jax.ShapeDtypeStruct((NP, H), jnp.float32),
                   jax.ShapeDtypeStruct((NBLK, 8, H), jnp.float32)],
    )(x32, S1, S1, S2, S2, S3, S3, dinv, a0, a1, a2, a3, bvec)


def _lbn_body(o_ref, st_ref, g_ref, be_ref, dv_ref, h_ref, u_ref, *, slope):
    st = jnp.sum(st_ref[...], axis=0)
    mu = st[0] / NREAL
    var = st[1] / NREAL - mu * mu
    xh = (o_ref[...] - mu) / jnp.sqrt(var + 1e-5) * g_ref[0] + be_ref[0]
    h = jnp.where(xh >= 0.0, xh, xh * slope)
    h_ref[...] = h
    u_ref[...] = h * dv_ref[...]


def _lbn(out, stats, g, be, dinv, slope):
    body = functools.partial(_lbn_body, slope=slope)
    return pl.pallas_call(
        body, grid=(NBLK,),
        in_specs=[_rowspec(H),
                  pl.BlockSpec((NBLK, 8, H), lambda j: (0, 0, 0)),
                  _fullspec((1, H)), _fullspec((1, H)), _rowspec(1)],
        out_specs=[_rowspec(H), _rowspec(H)],
        out_shape=[jax.ShapeDtypeStruct((NP, H), jnp.float32),
                   jax.ShapeDtypeStruct((NP, H), jnp.float32)],
    )(out, stats, g.reshape(1, H), be.reshape(1, H), dinv)


def _cheb_out(h_ref, schunks, dv, a0, a1, a2, a3, bv):
    dvv = dv[...]
    t0 = h_ref[...]
    out = jnp.dot(t0, a0[...], preferred_element_type=jnp.float32)
    w1, w2, w3 = a1[...], a2[...], a3[...]
    t1s, t2s, t3s = [], [], []
    for q in range(4):
        t1q = -schunks[0][q][0] * dvv
        t2q = -2.0 * schunks[1][q][0] * dvv - t0[:, 32 * q:32 * q + 32]
        t3q = -2.0 * schunks[2][q][0] * dvv - t1q
        t1s.append(t1q); t2s.append(t2q); t3s.append(t3q)
    for q in range(4):
        out += jnp.dot(t1s[q], w1[32 * q:32 * q + 32, :],
                       preferred_element_type=jnp.float32)
    for q in range(4):
        out += jnp.dot(t2s[q], w2[32 * q:32 * q + 32, :],
                       preferred_element_type=jnp.float32)
    for q in range(4):
        out += jnp.dot(t3s[q], w3[32 * q:32 * q + 32, :],
                       preferred_element_type=jnp.float32)
    return out + bv[...]


def _la_body(h_ref, s10, s11, s12, s13, s20, s21, s22, s23, s30, s31, s32, s33,
             dv, a0, a1, a2, a3, bv, o_ref, st_ref):
    j = pl.program_id(0)
    out = _cheb_out(h_ref, ((s10, s11, s12, s13), (s20, s21, s22, s23),
                            (s30, s31, s32, s33)), dv, a0, a1, a2, a3, bv)
    o_ref[...] = out
    st_ref[...] = _stats_of(out, j)


def _la(h, S1, S2, S3, dinv, a0, a1, a2, a3, bvec):
    cs = [_chunkspec(q) for q in range(4)]
    return pl.pallas_call(
        _la_body, grid=(NBLK,),
        in_specs=[_rowspec(H)] + cs + cs + cs
                 + [_rowspec(1)]
                 + [_fullspec((H, H))] * 4 + [_fullspec((1, H))],
        out_specs=[_rowspec(H), pl.BlockSpec((1, 8, H), lambda j: (j, 0, 0))],
        out_shape=[jax.ShapeDtypeStruct((NP, H), jnp.float32),
                   jax.ShapeDtypeStruct((NBLK, 8, H), jnp.float32)],
    )(h, S1, S1, S1, S1, S2, S2, S2, S2, S3, S3, S3, S3,
      dinv, a0, a1, a2, a3, bvec)


def _l4_body(h_ref, s10, s11, s12, s13, s20, s21, s22, s23, s30, s31, s32, s33,
             dv, a0, a1, a2, a3, bv, wr_ref, br_ref, o_ref):
    out = _cheb_out(h_ref, ((s10, s11, s12, s13), (s20, s21, s22, s23),
                            (s30, s31, s32, s33)), dv, a0, a1, a2, a3, bv)
    nrm = jnp.sqrt(jnp.sum(out * out, axis=1, keepdims=True))
    z = out / jnp.maximum(nrm, 1e-12)
    o_ref[...] = jnp.dot(z, wr_ref[...], preferred_element_type=jnp.float32) + br_ref[...]


def _l4(h, S1, S2, S3, dinv, a0, a1, a2, a3, bvec, Wr, br):
    cs = [_chunkspec(q) for q in range(4)]
    return pl.pallas_call(
        _l4_body, grid=(NBLK,),
        in_specs=[_rowspec(H)] + cs + cs + cs
                 + [_rowspec(1)]
                 + [_fullspec((H, H))] * 4 + [_fullspec((1, H))]
                 + [_fullspec((H, 3)), _fullspec((1, 3))],
        out_specs=_rowspec(3),
        out_shape=jax.ShapeDtypeStruct((NP, 3), jnp.float32),
    )(h, S1, S1, S1, S1, S2, S2, S2, S2, S3, S3, S3, S3,
      dinv, a0, a1, a2, a3, bvec, Wr, br.reshape(1, 3))


def _u2_body(s0, s1, s2, s3, d2_ref, o_ref):
    d2 = d2_ref[...]
    for q, sr in enumerate((s0, s1, s2, s3)):
        o_ref[:, 32 * q:32 * q + 32] = -sr[0] * d2


def _u2full(S1, dinv2):
    cs = [_chunkspec(q) for q in range(4)]
    return pl.pallas_call(
        _u2_body, grid=(NBLK,),
        in_specs=cs + [_rowspec(1)],
        out_specs=_rowspec(H),
        out_shape=jax.ShapeDtypeStruct((NP, H), jnp.float32),
    )(S1, S1, S1, S1, dinv2)


def _u3_body(s0, s1, s2, s3, u_ref, d2_ref, o_ref):
    d2 = d2_ref[...]
    u = u_ref[...]
    for q, sr in enumerate((s0, s1, s2, s3)):
        o_ref[:, 32 * q:32 * q + 32] = (-2.0 * sr[0] * d2
                                        - u[:, 32 * q:32 * q + 32])


def _u3full(S2, u1, dinv2):
    cs = [_chunkspec(q) for q in range(4)]
    return pl.pallas_call(
        _u3_body, grid=(NBLK,),
        in_specs=cs + [_rowspec(H), _rowspec(1)],
        out_specs=_rowspec(H),
        out_shape=jax.ShapeDtypeStruct((NP, H), jnp.float32),
    )(S2, S2, S2, S2, u1, dinv2)


def kernel(x, edge_index, W1, b1, g1, be1, W2, b2, g2, be2, W3, b3, g3, be3,
           W4, b4, Wr, br):
    f32 = jnp.float32
    E = edge_index.shape[1]
    src = edge_index[0].astype(jnp.int32)
    dst = edge_index[1].astype(jnp.int32)
    pe = EPAD - E
    ar = jnp.arange(pe, dtype=jnp.int32)
    srcp = jnp.concatenate([src, N + (ar % 48)])
    dstp = jnp.concatenate([dst, (ar * 2557) % NP])
    srcb = srcp.reshape(EB, 128)
    dstb = dstp.reshape(EB, 128)
    srcq = ((srcp * 4)[None, :]
            + jnp.arange(4, dtype=jnp.int32)[:, None]).reshape(4, EB, 128)
    x32 = jnp.zeros((NP, 32), f32).at[:N, :3].set(x)

    degp = _deg_call(srcb).reshape(2, NP)
    deg = degp[0] + degp[1]
    valid = jnp.arange(NP) < N
    dinv = jnp.where(valid & (deg > 0),
                     1.0 / jnp.sqrt(jnp.maximum(deg, 1e-12)), 0.0)
    dinv = dinv.astype(f32)[:, None]
    dinv2 = dinv * dinv

    uxv = _ux(x32, dinv)
    S1 = _xprop_call(uxv, srcb, dstb)
    u2 = _u2w3(S1, dinv2)
    S2 = _xprop_call(u2, srcb, dstb)
    u3 = _u3w3(S2, uxv, dinv2)
    S3 = _xprop_call(u3, srcb, dstb)
    pad32 = lambda w: jnp.zeros((32, H), f32).at[:3].set(w)
    out1, st1 = _l1a(x32, S1, S2, S3, dinv,
                     pad32(W1[0]), pad32(W1[1]), pad32(W1[2]), pad32(W1[3]),
                     b1.reshape(1, H))
    h, u1 = _lbn(out1, st1, g1, be1, dinv, 0.01)

    for (W, b, g, be, slope) in ((W2, b2, g2, be2, 0.01),
                                 (W3, b3, g3, be3, 0.0)):
        S1 = _prop_call(u1.reshape(NP * 4, 32), srcq, dstb)
        u2 = _u2full(S1, dinv2)
        S2 = _prop_call(u2.reshape(NP * 4, 32), srcq, dstb)
        u3 = _u3full(S2, u1, dinv2)
        S3 = _prop_call(u3.reshape(NP * 4, 32), srcq, dstb)
        out, st = _la(h, S1, S2, S3, dinv, W[0], W[1], W[2], W[3],
                      b.reshape(1, H))
        h, u1 = _lbn(out, st, g, be, dinv, slope)

    S1 = _prop_call(u1.reshape(NP * 4, 32), srcq, dstb)
    u2 = _u2full(S1, dinv2)
    S2 = _prop_call(u2.reshape(NP * 4, 32), srcq, dstb)
    u3 = _u3full(S2, u1, dinv2)
    S3 = _prop_call(u3.reshape(NP * 4, 32), srcq, dstb)
    final = _l4(h, S1, S2, S3, dinv, W4[0], W4[1], W4[2], W4[3],
                b4.reshape(1, H), Wr, br)
    return final[:N]
